# baseline (speedup 1.0000x reference)
import functools

import jax
import jax.numpy as jnp
from jax import lax
from jax.experimental import pallas as pl
from jax.experimental.pallas import tpu as pltpu
from jax.experimental.pallas import tpu_sc as plsc

_K = 16
_R2 = 16.0
_NC = 2
_NS = 16


def _pick_block(n, cap=512):
    best = 8
    for t in range(8, cap + 1, 8):
        if n % t == 0:
            best = t
    return best


def _mm(a, b):
    return lax.dot_general(a, b, (((1,), (0,)), ((), ())),
                           preferred_element_type=jnp.float32)


def _select_body(tgt_ref, posT_ref, srcM_ref, W1f_ref, W1p_ref,
                 idx_ref, val_ref, F_ref, d2_ref):
    @pl.when(pl.program_id(0) == 0)
    def _():
        srcM = srcM_ref[...]
        F_ref[...] = (_mm(srcM[:, :128], W1f_ref[...])
                      + _mm(srcM[:, 128:131], W1p_ref[...]))

    tgt = tgt_ref[...]
    posT = posT_ref[...]
    d2_ref[...] = ((tgt[:, 0:1] - posT[0:1, :]) ** 2
                   + (tgt[:, 1:2] - posT[1:2, :]) ** 2
                   + (tgt[:, 2:3] - posT[2:3, :]) ** 2)
    iota = lax.broadcasted_iota(jnp.int32, d2_ref.shape, 1)
    T = tgt.shape[0]
    lane_k = lax.broadcasted_iota(jnp.int32, (T, _K), 1)

    def step(k, tok):
        d2p = d2_ref[...]
        m = jnp.min(d2p, axis=1, keepdims=True)
        am = jnp.argmin(d2p, axis=1, keepdims=True)
        sel = lane_k == k
        idx_ref[...] = jnp.where(sel, am, idx_ref[...])
        val_ref[...] = jnp.where(
            sel, (m <= _R2).astype(jnp.float32), val_ref[...])
        d2_ref[...] = jnp.where(iota == am, jnp.inf, d2p)
        return tok

    lax.fori_loop(0, _K, step, 0)


def _select(target, posT, srcM, W1f, W1p, interpret=False):
    n_tgt = target.shape[0]
    n_src = posT.shape[1]
    T = _pick_block(n_tgt)
    fixed = lambda *shape: pl.BlockSpec(shape, lambda i: (0,) * len(shape))
    return pl.pallas_call(
        _select_body,
        grid=(n_tgt // T,),
        in_specs=[
            pl.BlockSpec((T, 3), lambda i: (i, 0)),
            fixed(3, n_src),
            fixed(n_src, srcM.shape[1]),
            fixed(128, 128),
            fixed(3, 128),
        ],
        out_specs=[
            pl.BlockSpec((T, _K), lambda i: (i, 0)),
            pl.BlockSpec((T, _K), lambda i: (i, 0)),
            fixed(n_src, 128),
        ],
        out_shape=[
            jax.ShapeDtypeStruct((n_tgt, _K), jnp.int32),
            jax.ShapeDtypeStruct((n_tgt, _K), jnp.float32),
            jax.ShapeDtypeStruct((n_src, 128), jnp.float32),
        ],
        scratch_shapes=[pltpu.VMEM((T, n_src), jnp.float32)],
        interpret=interpret,
    )(target, posT, srcM, W1f, W1p)


def _make_gather(n_edges, d):
    per_w = n_edges // (_NC * _NS)
    ch = 8
    for c in range(8, 129, 8):
        if per_w % c == 0:
            ch = c
    iters = per_w // ch
    mesh = plsc.VectorSubcoreMesh(core_axis_name="c", subcore_axis_name="s")

    @functools.partial(
        pl.kernel, mesh=mesh,
        out_type=jax.ShapeDtypeStruct((n_edges, d), jnp.float32),
        scratch_types=[
            pltpu.VMEM((per_w,), jnp.int32),
            pltpu.VMEM((ch, d), jnp.float32),
            pltpu.SemaphoreType.DMA,
        ],
    )
    def gather(idx_hbm, table_hbm, out_hbm, idx_v, rows_v, sem):
        wid = lax.axis_index("s") * _NC + lax.axis_index("c")
        base = pl.multiple_of(wid * per_w, 8)
        pltpu.sync_copy(idx_hbm.at[pl.ds(base, per_w)], idx_v)

        def step(j, tok):
            off = pl.multiple_of(j * ch, 8)
            pltpu.async_copy(
                table_hbm.at[idx_v.at[pl.ds(off, ch)]], rows_v, sem).wait()
            pltpu.sync_copy(rows_v, out_hbm.at[pl.ds(base + off, ch)])
            return tok

        lax.fori_loop(0, iters, step, 0)

    return gather


def _mlp_body(tgt_ref, gath_ref, val_ref, src_ref, W1p_ref, b1_ref,
              W2_ref, b2_ref, W3_ref, b3_ref,
              pooled_ref, srccopy_ref, anyv_ref):
    tgt = tgt_ref[...]
    T = tgt.shape[0]
    d_out = b3_ref.shape[-1]
    tW1p = _mm(tgt, W1p_ref[...])
    pooled_ref[...] = jnp.full((T, d_out), -jnp.inf, jnp.float32)
    anyv_ref[...] = jnp.zeros((T, 1), jnp.float32)
    for k in range(_K):
        g = gath_ref[k]
        h = jax.nn.relu(g - tW1p + b1_ref[...])
        h = jax.nn.relu(_mm(h, W2_ref[...]) + b2_ref[...])
        h = jax.nn.relu(_mm(h, W3_ref[...]) + b3_ref[...])
        valid = val_ref[:, k:k + 1] > 0.0
        pooled_ref[...] = jnp.maximum(pooled_ref[...],
                                      jnp.where(valid, h, -jnp.inf))
        anyv_ref[...] = jnp.maximum(anyv_ref[...],
                                    valid.astype(jnp.float32))
    pooled_ref[...] = jnp.where(anyv_ref[...] > 0.0, pooled_ref[...], 0.0)

    @pl.when(pl.program_id(0) == 0)
    def _():
        srccopy_ref[...] = src_ref[...]


def _mlp_pool(target, gath, val, src, W1p, b1r, W2, b2r, W3, b3r,
              interpret=False):
    n_tgt = target.shape[0]
    n_src, width = src.shape
    d_out = b3r.shape[-1]
    T = _pick_block(n_tgt)
    fixed = lambda *shape: pl.BlockSpec(shape, lambda i: (0,) * len(shape))
    return pl.pallas_call(
        _mlp_body,
        grid=(n_tgt // T,),
        in_specs=[
            pl.BlockSpec((T, 3), lambda i: (i, 0)),
            pl.BlockSpec((_K, T, 128), lambda i: (0, i, 0)),
            pl.BlockSpec((T, _K), lambda i: (i, 0)),
            fixed(n_src, width),
            fixed(3, 128),
            fixed(1, 128),
            fixed(*W2.shape),
            fixed(1, W2.shape[1]),
            fixed(*W3.shape),
            fixed(1, d_out),
        ],
        out_specs=[
            pl.BlockSpec((T, d_out), lambda i: (i, 0)),
            fixed(n_src, width),
        ],
        out_shape=[
            jax.ShapeDtypeStruct((n_tgt, d_out), jnp.float32),
            jax.ShapeDtypeStruct((n_src, width), jnp.float32),
        ],
        scratch_shapes=[pltpu.VMEM((T, 1), jnp.float32)],
        interpret=interpret,
    )(target, gath, val, src, W1p, b1r, W2, b2r, W3, b3r)


def _forward(src, target, W1, b1, W2, b2, W3, b3):
    n_tgt = target.shape[0]
    posT = jnp.transpose(src[:, :3])
    srcM = jnp.concatenate([src[:, 3:], src[:, :3]], axis=1)
    W1f = W1[:128, :]
    W1p = W1[128:, :]

    idx, val, F = _select(target, posT, srcM, W1f, W1p)
    idx_kmajor = jnp.transpose(idx).reshape(-1)
    gath_flat = _make_gather(_K * n_tgt, 128)(idx_kmajor, F)
    gath = gath_flat.reshape(_K, n_tgt, 128)

    return _mlp_pool(target, gath, val, src, W1p, b1.reshape(1, -1),
                     W2, b2.reshape(1, -1), W3, b3.reshape(1, -1))


def kernel(src, target, W1, b1, W2, b2, W3, b3):
    _, src_out = _forward(src, target, W1, b1, W2, b2, W3, b3)
    return src_out


# speedup vs baseline: 1.1643x; 1.1643x over previous
import functools

import jax
import jax.numpy as jnp
from jax import lax
from jax.experimental import pallas as pl
from jax.experimental.pallas import tpu as pltpu
from jax.experimental.pallas import tpu_sc as plsc

_K = 16
_R2 = 16.0
_NC = 2
_NS = 16


def _pick_block(n, cap=512):
    best = 8
    for t in range(8, cap + 1, 8):
        if n % t == 0:
            best = t
    return best


def _mm(a, b):
    return lax.dot_general(a, b, (((1,), (0,)), ((), ())),
                           preferred_element_type=jnp.float32)


def _select_body(tgt_ref, posT_ref, srcM_ref, W1f_ref, W1p_ref,
                 idx_ref, val_ref, F_ref, d2_ref):
    @pl.when(pl.program_id(0) == 0)
    def _():
        srcM = srcM_ref[...]
        F_ref[...] = (_mm(srcM[:, :128], W1f_ref[...])
                      + _mm(srcM[:, 128:131], W1p_ref[...]))

    tgt = tgt_ref[...]
    posT = posT_ref[...]
    d2 = ((tgt[:, 0:1] - posT[0:1, :]) ** 2
          + (tgt[:, 1:2] - posT[1:2, :]) ** 2
          + (tgt[:, 2:3] - posT[2:3, :]) ** 2)
    d2_ref[...] = d2
    iota = lax.broadcasted_iota(jnp.int32, d2.shape, 1)
    T = tgt.shape[0]
    lane_k = lax.broadcasted_iota(jnp.int32, (T, _K), 1)
    nvalid = jnp.sum((d2 <= _R2).astype(jnp.float32), axis=1,
                     keepdims=True)
    val_ref[...] = (lane_k.astype(jnp.float32) < nvalid).astype(jnp.float32)

    def step(k, tok):
        d2p = d2_ref[...]
        am = jnp.argmin(d2p, axis=1, keepdims=True)
        idx_ref[...] = jnp.where(lane_k == k, am, idx_ref[...])
        d2_ref[...] = jnp.where(iota == am, jnp.inf, d2p)
        return tok

    lax.fori_loop(0, _K, step, 0)


def _select(target, posT, srcM, W1f, W1p, interpret=False):
    n_tgt = target.shape[0]
    n_src = posT.shape[1]
    T = _pick_block(n_tgt)
    fixed = lambda *shape: pl.BlockSpec(shape, lambda i: (0,) * len(shape))
    return pl.pallas_call(
        _select_body,
        grid=(n_tgt // T,),
        in_specs=[
            pl.BlockSpec((T, 3), lambda i: (i, 0)),
            fixed(3, n_src),
            fixed(n_src, srcM.shape[1]),
            fixed(128, 128),
            fixed(3, 128),
        ],
        out_specs=[
            pl.BlockSpec((T, _K), lambda i: (i, 0)),
            pl.BlockSpec((T, _K), lambda i: (i, 0)),
            fixed(n_src, 128),
        ],
        out_shape=[
            jax.ShapeDtypeStruct((n_tgt, _K), jnp.int32),
            jax.ShapeDtypeStruct((n_tgt, _K), jnp.float32),
            jax.ShapeDtypeStruct((n_src, 128), jnp.float32),
        ],
        scratch_shapes=[pltpu.VMEM((T, n_src), jnp.float32)],
        interpret=interpret,
    )(target, posT, srcM, W1f, W1p)


def _make_gather(n_edges, d):
    per_w = n_edges // (_NC * _NS)
    ch = 8
    for c in range(8, 129, 8):
        if per_w % c == 0:
            ch = c
    iters = per_w // ch
    mesh = plsc.VectorSubcoreMesh(core_axis_name="c", subcore_axis_name="s")

    @functools.partial(
        pl.kernel, mesh=mesh,
        out_type=jax.ShapeDtypeStruct((n_edges, d), jnp.float32),
        scratch_types=[
            pltpu.VMEM((per_w,), jnp.int32),
            pltpu.VMEM((ch, d), jnp.float32),
            pltpu.SemaphoreType.DMA,
        ],
    )
    def gather(idx_hbm, table_hbm, out_hbm, idx_v, rows_v, sem):
        wid = lax.axis_index("s") * _NC + lax.axis_index("c")
        base = pl.multiple_of(wid * per_w, 8)
        pltpu.sync_copy(idx_hbm.at[pl.ds(base, per_w)], idx_v)

        def step(j, tok):
            off = pl.multiple_of(j * ch, 8)
            pltpu.async_copy(
                table_hbm.at[idx_v.at[pl.ds(off, ch)]], rows_v, sem).wait()
            pltpu.sync_copy(rows_v, out_hbm.at[pl.ds(base + off, ch)])
            return tok

        lax.fori_loop(0, iters, step, 0)

    return gather


def _mlp_body(tgt_ref, gath_ref, val_ref, src_ref, W1p_ref, b1_ref,
              W2_ref, b2_ref, W3_ref, b3_ref,
              pooled_ref, srccopy_ref, anyv_ref):
    tgt = tgt_ref[...]
    T = tgt.shape[0]
    d_out = b3_ref.shape[-1]
    tW1p = _mm(tgt, W1p_ref[...])
    pooled_ref[...] = jnp.full((T, d_out), -jnp.inf, jnp.float32)
    anyv_ref[...] = jnp.zeros((T, 1), jnp.float32)
    for k in range(_K):
        g = gath_ref[k]
        h = jax.nn.relu(g - tW1p + b1_ref[...])
        h = jax.nn.relu(_mm(h, W2_ref[...]) + b2_ref[...])
        h = jax.nn.relu(_mm(h, W3_ref[...]) + b3_ref[...])
        valid = val_ref[:, k:k + 1] > 0.0
        pooled_ref[...] = jnp.maximum(pooled_ref[...],
                                      jnp.where(valid, h, -jnp.inf))
        anyv_ref[...] = jnp.maximum(anyv_ref[...],
                                    valid.astype(jnp.float32))
    pooled_ref[...] = jnp.where(anyv_ref[...] > 0.0, pooled_ref[...], 0.0)

    @pl.when(pl.program_id(0) == 0)
    def _():
        srccopy_ref[...] = src_ref[...]


def _mlp_pool(target, gath, val, src, W1p, b1r, W2, b2r, W3, b3r,
              interpret=False):
    n_tgt = target.shape[0]
    n_src, width = src.shape
    d_out = b3r.shape[-1]
    T = _pick_block(n_tgt)
    fixed = lambda *shape: pl.BlockSpec(shape, lambda i: (0,) * len(shape))
    return pl.pallas_call(
        _mlp_body,
        grid=(n_tgt // T,),
        in_specs=[
            pl.BlockSpec((T, 3), lambda i: (i, 0)),
            pl.BlockSpec((_K, T, 128), lambda i: (0, i, 0)),
            pl.BlockSpec((T, _K), lambda i: (i, 0)),
            fixed(n_src, width),
            fixed(3, 128),
            fixed(1, 128),
            fixed(*W2.shape),
            fixed(1, W2.shape[1]),
            fixed(*W3.shape),
            fixed(1, d_out),
        ],
        out_specs=[
            pl.BlockSpec((T, d_out), lambda i: (i, 0)),
            fixed(n_src, width),
        ],
        out_shape=[
            jax.ShapeDtypeStruct((n_tgt, d_out), jnp.float32),
            jax.ShapeDtypeStruct((n_src, width), jnp.float32),
        ],
        scratch_shapes=[pltpu.VMEM((T, 1), jnp.float32)],
        interpret=interpret,
    )(target, gath, val, src, W1p, b1r, W2, b2r, W3, b3r)


def _forward(src, target, W1, b1, W2, b2, W3, b3):
    n_tgt = target.shape[0]
    posT = jnp.transpose(src[:, :3])
    srcM = jnp.concatenate([src[:, 3:], src[:, :3]], axis=1)
    W1f = W1[:128, :]
    W1p = W1[128:, :]

    idx, val, F = _select(target, posT, srcM, W1f, W1p)
    idx_kmajor = jnp.transpose(idx).reshape(-1)
    gath_flat = _make_gather(_K * n_tgt, 128)(idx_kmajor, F)
    gath = gath_flat.reshape(_K, n_tgt, 128)

    return _mlp_pool(target, gath, val, src, W1p, b1.reshape(1, -1),
                     W2, b2.reshape(1, -1), W3, b3.reshape(1, -1))


def kernel(src, target, W1, b1, W2, b2, W3, b3):
    _, src_out = _forward(src, target, W1, b1, W2, b2, W3, b3)
    return src_out


# speedup vs baseline: 1.2284x; 1.0551x over previous
import functools

import jax
import jax.numpy as jnp
from jax import lax
from jax.experimental import pallas as pl
from jax.experimental.pallas import tpu as pltpu
from jax.experimental.pallas import tpu_sc as plsc

_K = 16
_R2 = 16.0
_NC = 2
_NS = 16


def _pick_block(n, cap=512):
    best = 8
    for t in range(8, cap + 1, 8):
        if n % t == 0:
            best = t
    return best


def _mm(a, b):
    return lax.dot_general(a, b, (((1,), (0,)), ((), ())),
                           preferred_element_type=jnp.float32)


def _select_body(tgt_ref, posT_ref, srcM_ref, W1f_ref, W1p_ref,
                 idx_ref, val_ref, F_ref, d2_ref):
    @pl.when(pl.program_id(0) == 0)
    def _():
        srcM = srcM_ref[...]
        F_ref[...] = (_mm(srcM[:, :128], W1f_ref[...])
                      + _mm(srcM[:, 128:131], W1p_ref[...]))

    tgt = tgt_ref[...]
    posT = posT_ref[...]
    d2 = ((tgt[:, 0:1] - posT[0:1, :]) ** 2
          + (tgt[:, 1:2] - posT[1:2, :]) ** 2
          + (tgt[:, 2:3] - posT[2:3, :]) ** 2)
    d2_ref[...] = d2
    iota = lax.broadcasted_iota(jnp.int32, d2.shape, 1)
    T = tgt.shape[0]
    lane_k = lax.broadcasted_iota(jnp.int32, (T, _K), 1)
    nvalid = jnp.sum((d2 <= _R2).astype(jnp.float32), axis=1,
                     keepdims=True)
    val_ref[...] = (lane_k.astype(jnp.float32) < nvalid).astype(jnp.float32)

    big = jnp.int32(1 << 30)

    def step(k, tok):
        d2p = d2_ref[...]
        m = jnp.min(d2p, axis=1, keepdims=True)
        am = jnp.min(jnp.where(d2p == m, iota, big), axis=1,
                     keepdims=True)
        idx_ref[...] = jnp.where(lane_k == k, am, idx_ref[...])
        d2_ref[...] = jnp.where(iota == am, jnp.inf, d2p)
        return tok

    lax.fori_loop(0, _K, step, 0)


def _select(target, posT, srcM, W1f, W1p, interpret=False):
    n_tgt = target.shape[0]
    n_src = posT.shape[1]
    T = _pick_block(n_tgt)
    fixed = lambda *shape: pl.BlockSpec(shape, lambda i: (0,) * len(shape))
    return pl.pallas_call(
        _select_body,
        grid=(n_tgt // T,),
        in_specs=[
            pl.BlockSpec((T, 3), lambda i: (i, 0)),
            fixed(3, n_src),
            fixed(n_src, srcM.shape[1]),
            fixed(128, 128),
            fixed(3, 128),
        ],
        out_specs=[
            pl.BlockSpec((T, _K), lambda i: (i, 0)),
            pl.BlockSpec((T, _K), lambda i: (i, 0)),
            fixed(n_src, 128),
        ],
        out_shape=[
            jax.ShapeDtypeStruct((n_tgt, _K), jnp.int32),
            jax.ShapeDtypeStruct((n_tgt, _K), jnp.float32),
            jax.ShapeDtypeStruct((n_src, 128), jnp.float32),
        ],
        scratch_shapes=[pltpu.VMEM((T, n_src), jnp.float32)],
        interpret=interpret,
    )(target, posT, srcM, W1f, W1p)


def _make_gather(n_edges, d):
    per_w = n_edges // (_NC * _NS)
    ch = 8
    for c in range(8, 129, 8):
        if per_w % c == 0:
            ch = c
    iters = per_w // ch
    mesh = plsc.VectorSubcoreMesh(core_axis_name="c", subcore_axis_name="s")

    @functools.partial(
        pl.kernel, mesh=mesh,
        out_type=jax.ShapeDtypeStruct((n_edges, d), jnp.float32),
        scratch_types=[
            pltpu.VMEM((per_w,), jnp.int32),
            pltpu.VMEM((ch, d), jnp.float32),
            pltpu.SemaphoreType.DMA,
        ],
    )
    def gather(idx_hbm, table_hbm, out_hbm, idx_v, rows_v, sem):
        wid = lax.axis_index("s") * _NC + lax.axis_index("c")
        base = pl.multiple_of(wid * per_w, 8)
        pltpu.sync_copy(idx_hbm.at[pl.ds(base, per_w)], idx_v)

        def step(j, tok):
            off = pl.multiple_of(j * ch, 8)
            pltpu.async_copy(
                table_hbm.at[idx_v.at[pl.ds(off, ch)]], rows_v, sem).wait()
            pltpu.sync_copy(rows_v, out_hbm.at[pl.ds(base + off, ch)])
            return tok

        lax.fori_loop(0, iters, step, 0)

    return gather


def _mlp_body(tgt_ref, gath_ref, val_ref, src_ref, W1p_ref, b1_ref,
              W2_ref, b2_ref, W3_ref, b3_ref,
              pooled_ref, srccopy_ref, anyv_ref):
    tgt = tgt_ref[...]
    T = tgt.shape[0]
    d_out = b3_ref.shape[-1]
    tW1p = _mm(tgt, W1p_ref[...])
    pooled_ref[...] = jnp.full((T, d_out), -jnp.inf, jnp.float32)
    anyv_ref[...] = jnp.zeros((T, 1), jnp.float32)
    for k in range(_K):
        g = gath_ref[k]
        h = jax.nn.relu(g - tW1p + b1_ref[...])
        h = jax.nn.relu(_mm(h, W2_ref[...]) + b2_ref[...])
        h = jax.nn.relu(_mm(h, W3_ref[...]) + b3_ref[...])
        valid = val_ref[:, k:k + 1] > 0.0
        pooled_ref[...] = jnp.maximum(pooled_ref[...],
                                      jnp.where(valid, h, -jnp.inf))
        anyv_ref[...] = jnp.maximum(anyv_ref[...],
                                    valid.astype(jnp.float32))
    pooled_ref[...] = jnp.where(anyv_ref[...] > 0.0, pooled_ref[...], 0.0)

    @pl.when(pl.program_id(0) == 0)
    def _():
        srccopy_ref[...] = src_ref[...]


def _mlp_pool(target, gath, val, src, W1p, b1r, W2, b2r, W3, b3r,
              interpret=False):
    n_tgt = target.shape[0]
    n_src, width = src.shape
    d_out = b3r.shape[-1]
    T = _pick_block(n_tgt)
    fixed = lambda *shape: pl.BlockSpec(shape, lambda i: (0,) * len(shape))
    return pl.pallas_call(
        _mlp_body,
        grid=(n_tgt // T,),
        in_specs=[
            pl.BlockSpec((T, 3), lambda i: (i, 0)),
            pl.BlockSpec((_K, T, 128), lambda i: (0, i, 0)),
            pl.BlockSpec((T, _K), lambda i: (i, 0)),
            fixed(n_src, width),
            fixed(3, 128),
            fixed(1, 128),
            fixed(*W2.shape),
            fixed(1, W2.shape[1]),
            fixed(*W3.shape),
            fixed(1, d_out),
        ],
        out_specs=[
            pl.BlockSpec((T, d_out), lambda i: (i, 0)),
            fixed(n_src, width),
        ],
        out_shape=[
            jax.ShapeDtypeStruct((n_tgt, d_out), jnp.float32),
            jax.ShapeDtypeStruct((n_src, width), jnp.float32),
        ],
        scratch_shapes=[pltpu.VMEM((T, 1), jnp.float32)],
        interpret=interpret,
    )(target, gath, val, src, W1p, b1r, W2, b2r, W3, b3r)


def _forward(src, target, W1, b1, W2, b2, W3, b3):
    n_tgt = target.shape[0]
    posT = jnp.transpose(src[:, :3])
    srcM = jnp.concatenate([src[:, 3:], src[:, :3]], axis=1)
    W1f = W1[:128, :]
    W1p = W1[128:, :]

    idx, val, F = _select(target, posT, srcM, W1f, W1p)
    idx_kmajor = jnp.transpose(idx).reshape(-1)
    gath_flat = _make_gather(_K * n_tgt, 128)(idx_kmajor, F)
    gath = gath_flat.reshape(_K, n_tgt, 128)

    return _mlp_pool(target, gath, val, src, W1p, b1.reshape(1, -1),
                     W2, b2.reshape(1, -1), W3, b3.reshape(1, -1))


def kernel(src, target, W1, b1, W2, b2, W3, b3):
    _, src_out = _forward(src, target, W1, b1, W2, b2, W3, b3)
    return src_out


# speedup vs baseline: 1.2356x; 1.0059x over previous
import functools

import jax
import jax.numpy as jnp
from jax import lax
from jax.experimental import pallas as pl
from jax.experimental.pallas import tpu as pltpu
from jax.experimental.pallas import tpu_sc as plsc

_K = 16
_R2 = 16.0
_NC = 2
_NS = 16


def _pick_block(n, cap=512):
    best = 8
    for t in range(8, cap + 1, 8):
        if n % t == 0:
            best = t
    return best


def _mm(a, b):
    return lax.dot_general(a, b, (((1,), (0,)), ((), ())),
                           preferred_element_type=jnp.float32)


def _select_body(tgt_ref, posT_ref, srcM_ref, W1f_ref, W1p_ref,
                 idx_ref, val_ref, F_ref, d2_ref):
    @pl.when(pl.program_id(0) == 0)
    def _():
        srcM = srcM_ref[...]
        F_ref[...] = (_mm(srcM[:, :128], W1f_ref[...])
                      + _mm(srcM[:, 128:131], W1p_ref[...]))

    tgt = tgt_ref[...]
    posT = posT_ref[...]
    d2 = ((tgt[:, 0:1] - posT[0:1, :]) ** 2
          + (tgt[:, 1:2] - posT[1:2, :]) ** 2
          + (tgt[:, 2:3] - posT[2:3, :]) ** 2)
    d2_ref[...] = d2
    iota = lax.broadcasted_iota(jnp.int32, d2.shape, 1)
    T = tgt.shape[0]
    lane_k = lax.broadcasted_iota(jnp.int32, (T, _K), 1)
    nvalid = jnp.sum((d2 <= _R2).astype(jnp.float32), axis=1,
                     keepdims=True)
    val_ref[...] = (lane_k.astype(jnp.float32) < nvalid).astype(jnp.float32)

    big = jnp.int32(1 << 30)

    def extract(d2p):
        m = jnp.min(d2p, axis=1, keepdims=True)
        am = jnp.min(jnp.where(d2p == m, iota, big), axis=1,
                     keepdims=True)
        return am, jnp.where(iota == am, jnp.inf, d2p)

    def step(h, tok):
        am1, d2m = extract(d2_ref[...])
        am2, d2m = extract(d2m)
        idx_ref[...] = jnp.where(
            lane_k == 2 * h, am1,
            jnp.where(lane_k == 2 * h + 1, am2, idx_ref[...]))
        d2_ref[...] = d2m
        return tok

    lax.fori_loop(0, _K // 2, step, 0)


def _select(target, posT, srcM, W1f, W1p, interpret=False):
    n_tgt = target.shape[0]
    n_src = posT.shape[1]
    T = _pick_block(n_tgt)
    fixed = lambda *shape: pl.BlockSpec(shape, lambda i: (0,) * len(shape))
    return pl.pallas_call(
        _select_body,
        grid=(n_tgt // T,),
        in_specs=[
            pl.BlockSpec((T, 3), lambda i: (i, 0)),
            fixed(3, n_src),
            fixed(n_src, srcM.shape[1]),
            fixed(128, 128),
            fixed(3, 128),
        ],
        out_specs=[
            pl.BlockSpec((T, _K), lambda i: (i, 0)),
            pl.BlockSpec((T, _K), lambda i: (i, 0)),
            fixed(n_src, 128),
        ],
        out_shape=[
            jax.ShapeDtypeStruct((n_tgt, _K), jnp.int32),
            jax.ShapeDtypeStruct((n_tgt, _K), jnp.float32),
            jax.ShapeDtypeStruct((n_src, 128), jnp.float32),
        ],
        scratch_shapes=[pltpu.VMEM((T, n_src), jnp.float32)],
        interpret=interpret,
    )(target, posT, srcM, W1f, W1p)


def _make_gather(n_edges, d):
    per_w = n_edges // (_NC * _NS)
    ch = 8
    for c in range(8, 129, 8):
        if per_w % c == 0:
            ch = c
    iters = per_w // ch
    mesh = plsc.VectorSubcoreMesh(core_axis_name="c", subcore_axis_name="s")

    @functools.partial(
        pl.kernel, mesh=mesh,
        out_type=jax.ShapeDtypeStruct((n_edges, d), jnp.float32),
        scratch_types=[
            pltpu.VMEM((per_w,), jnp.int32),
            pltpu.VMEM((ch, d), jnp.float32),
            pltpu.SemaphoreType.DMA,
        ],
    )
    def gather(idx_hbm, table_hbm, out_hbm, idx_v, rows_v, sem):
        wid = lax.axis_index("s") * _NC + lax.axis_index("c")
        base = pl.multiple_of(wid * per_w, 8)
        pltpu.sync_copy(idx_hbm.at[pl.ds(base, per_w)], idx_v)

        def step(j, tok):
            off = pl.multiple_of(j * ch, 8)
            pltpu.async_copy(
                table_hbm.at[idx_v.at[pl.ds(off, ch)]], rows_v, sem).wait()
            pltpu.sync_copy(rows_v, out_hbm.at[pl.ds(base + off, ch)])
            return tok

        lax.fori_loop(0, iters, step, 0)

    return gather


def _mlp_body(tgt_ref, gath_ref, val_ref, src_ref, W1p_ref, b1_ref,
              W2_ref, b2_ref, W3_ref, b3_ref,
              pooled_ref, srccopy_ref, anyv_ref):
    tgt = tgt_ref[...]
    T = tgt.shape[0]
    d_out = b3_ref.shape[-1]
    tW1p = _mm(tgt, W1p_ref[...])
    pooled_ref[...] = jnp.full((T, d_out), -jnp.inf, jnp.float32)
    anyv_ref[...] = jnp.zeros((T, 1), jnp.float32)
    for k in range(_K):
        g = gath_ref[k]
        h = jax.nn.relu(g - tW1p + b1_ref[...])
        h = jax.nn.relu(_mm(h, W2_ref[...]) + b2_ref[...])
        h = jax.nn.relu(_mm(h, W3_ref[...]) + b3_ref[...])
        valid = val_ref[:, k:k + 1] > 0.0
        pooled_ref[...] = jnp.maximum(pooled_ref[...],
                                      jnp.where(valid, h, -jnp.inf))
        anyv_ref[...] = jnp.maximum(anyv_ref[...],
                                    valid.astype(jnp.float32))
    pooled_ref[...] = jnp.where(anyv_ref[...] > 0.0, pooled_ref[...], 0.0)

    @pl.when(pl.program_id(0) == 0)
    def _():
        srccopy_ref[...] = src_ref[...]


def _mlp_pool(target, gath, val, src, W1p, b1r, W2, b2r, W3, b3r,
              interpret=False):
    n_tgt = target.shape[0]
    n_src, width = src.shape
    d_out = b3r.shape[-1]
    T = _pick_block(n_tgt)
    fixed = lambda *shape: pl.BlockSpec(shape, lambda i: (0,) * len(shape))
    return pl.pallas_call(
        _mlp_body,
        grid=(n_tgt // T,),
        in_specs=[
            pl.BlockSpec((T, 3), lambda i: (i, 0)),
            pl.BlockSpec((_K, T, 128), lambda i: (0, i, 0)),
            pl.BlockSpec((T, _K), lambda i: (i, 0)),
            fixed(n_src, width),
            fixed(3, 128),
            fixed(1, 128),
            fixed(*W2.shape),
            fixed(1, W2.shape[1]),
            fixed(*W3.shape),
            fixed(1, d_out),
        ],
        out_specs=[
            pl.BlockSpec((T, d_out), lambda i: (i, 0)),
            fixed(n_src, width),
        ],
        out_shape=[
            jax.ShapeDtypeStruct((n_tgt, d_out), jnp.float32),
            jax.ShapeDtypeStruct((n_src, width), jnp.float32),
        ],
        scratch_shapes=[pltpu.VMEM((T, 1), jnp.float32)],
        interpret=interpret,
    )(target, gath, val, src, W1p, b1r, W2, b2r, W3, b3r)


def _forward(src, target, W1, b1, W2, b2, W3, b3):
    n_tgt = target.shape[0]
    posT = jnp.transpose(src[:, :3])
    srcM = jnp.concatenate([src[:, 3:], src[:, :3]], axis=1)
    W1f = W1[:128, :]
    W1p = W1[128:, :]

    idx, val, F = _select(target, posT, srcM, W1f, W1p)
    idx_kmajor = jnp.transpose(idx).reshape(-1)
    gath_flat = _make_gather(_K * n_tgt, 128)(idx_kmajor, F)
    gath = gath_flat.reshape(_K, n_tgt, 128)

    return _mlp_pool(target, gath, val, src, W1p, b1.reshape(1, -1),
                     W2, b2.reshape(1, -1), W3, b3.reshape(1, -1))


def kernel(src, target, W1, b1, W2, b2, W3, b3):
    _, src_out = _forward(src, target, W1, b1, W2, b2, W3, b3)
    return src_out


# speedup vs baseline: 1.2479x; 1.0099x over previous
import functools

import jax
import jax.numpy as jnp
from jax import lax
from jax.experimental import pallas as pl
from jax.experimental.pallas import tpu as pltpu
from jax.experimental.pallas import tpu_sc as plsc

_K = 16
_R2 = 16.0
_NC = 2
_NS = 16


def _pick_block(n, cap=512):
    best = 8
    for t in range(8, cap + 1, 8):
        if n % t == 0:
            best = t
    return best


def _mm(a, b):
    return lax.dot_general(a, b, (((1,), (0,)), ((), ())),
                           preferred_element_type=jnp.float32)


def _select_body(tgt_ref, posT_ref, srcM_ref, W1f_ref, W1p_ref,
                 idx_ref, val_ref, F_ref, d2_ref):
    @pl.when(pl.program_id(0) == 0)
    def _():
        srcM = srcM_ref[...]
        F_ref[...] = (_mm(srcM[:, :128], W1f_ref[...])
                      + _mm(srcM[:, 128:131], W1p_ref[...]))

    tgt = tgt_ref[...]
    posT = posT_ref[...]
    d2 = ((tgt[:, 0:1] - posT[0:1, :]) ** 2
          + (tgt[:, 1:2] - posT[1:2, :]) ** 2
          + (tgt[:, 2:3] - posT[2:3, :]) ** 2)
    d2_ref[...] = d2
    iota = lax.broadcasted_iota(jnp.int32, d2.shape, 1)
    T = tgt.shape[0]
    lane_k = lax.broadcasted_iota(jnp.int32, (T, _K), 1)
    nvalid = jnp.sum((d2 <= _R2).astype(jnp.float32), axis=1,
                     keepdims=True)
    val_ref[...] = (lane_k.astype(jnp.float32) < nvalid).astype(jnp.float32)

    big = jnp.int32(1 << 30)

    def extract(d2p):
        m = jnp.min(d2p, axis=1, keepdims=True)
        am = jnp.min(jnp.where(d2p == m, iota, big), axis=1,
                     keepdims=True)
        return am, jnp.where(iota == am, jnp.inf, d2p)

    def step(h, tok):
        am1, d2m = extract(d2_ref[...])
        am2, d2m = extract(d2m)
        idx_ref[...] = jnp.where(
            lane_k == 2 * h, am1,
            jnp.where(lane_k == 2 * h + 1, am2, idx_ref[...]))
        d2_ref[...] = d2m
        return tok

    lax.fori_loop(0, _K // 2, step, 0)


def _select(target, posT, srcM, W1f, W1p):
    n_tgt = target.shape[0]
    n_src = posT.shape[1]
    T = _pick_block(n_tgt)
    fixed = lambda *shape: pl.BlockSpec(shape, lambda i: (0,) * len(shape))
    return pl.pallas_call(
        _select_body,
        grid=(n_tgt // T,),
        in_specs=[
            pl.BlockSpec((T, 3), lambda i: (i, 0)),
            fixed(3, n_src),
            fixed(n_src, srcM.shape[1]),
            fixed(128, 128),
            fixed(3, 128),
        ],
        out_specs=[
            pl.BlockSpec((T, _K), lambda i: (i, 0)),
            pl.BlockSpec((T, _K), lambda i: (i, 0)),
            fixed(n_src, 128),
        ],
        out_shape=[
            jax.ShapeDtypeStruct((n_tgt, _K), jnp.int32),
            jax.ShapeDtypeStruct((n_tgt, _K), jnp.float32),
            jax.ShapeDtypeStruct((n_src, 128), jnp.float32),
        ],
        scratch_shapes=[pltpu.VMEM((T, n_src), jnp.float32)],
    )(target, posT, srcM, W1f, W1p)


def _make_gather(n_edges, d):
    per_w = n_edges // (_NC * _NS)
    ch = 8
    for c in range(8, 129, 8):
        if per_w % c == 0:
            ch = c
    iters = per_w // ch
    mesh = plsc.VectorSubcoreMesh(core_axis_name="c", subcore_axis_name="s")

    @functools.partial(
        pl.kernel, mesh=mesh,
        out_type=jax.ShapeDtypeStruct((n_edges, d), jnp.float32),
        scratch_types=[
            pltpu.VMEM((per_w,), jnp.int32),
            pltpu.VMEM((ch, d), jnp.float32),
            pltpu.SemaphoreType.DMA,
        ],
    )
    def gather(idx_hbm, table_hbm, out_hbm, idx_v, rows_v, sem):
        wid = lax.axis_index("s") * _NC + lax.axis_index("c")
        base = pl.multiple_of(wid * per_w, 8)
        pltpu.sync_copy(idx_hbm.at[pl.ds(base, per_w)], idx_v)

        def step(j, tok):
            off = pl.multiple_of(j * ch, 8)
            pltpu.async_copy(
                table_hbm.at[idx_v.at[pl.ds(off, ch)]], rows_v, sem).wait()
            pltpu.sync_copy(rows_v, out_hbm.at[pl.ds(base + off, ch)])
            return tok

        lax.fori_loop(0, iters, step, 0)

    return gather


def _mlp_body(tgt_ref, gath_ref, val_ref, src_ref, W1p_ref, b1_ref,
              W2_ref, b2_ref, W3_ref, b3_ref,
              pooled_ref, srccopy_ref, anyv_ref):
    tgt = tgt_ref[...]
    T = tgt.shape[0]
    d_out = b3_ref.shape[-1]
    tW1p = _mm(tgt, W1p_ref[...])
    pooled_ref[...] = jnp.full((T, d_out), -jnp.inf, jnp.float32)
    anyv_ref[...] = jnp.zeros((T, 1), jnp.float32)
    for k in range(_K):
        g = gath_ref[k]
        h = jax.nn.relu(g - tW1p + b1_ref[...])
        h = jax.nn.relu(_mm(h, W2_ref[...]) + b2_ref[...])
        h = jax.nn.relu(_mm(h, W3_ref[...]) + b3_ref[...])
        valid = val_ref[:, k:k + 1] > 0.0
        pooled_ref[...] = jnp.maximum(pooled_ref[...],
                                      jnp.where(valid, h, -jnp.inf))
        anyv_ref[...] = jnp.maximum(anyv_ref[...],
                                    valid.astype(jnp.float32))
    pooled_ref[...] = jnp.where(anyv_ref[...] > 0.0, pooled_ref[...], 0.0)

    @pl.when(pl.program_id(0) == 0)
    def _():
        srccopy_ref[...] = src_ref[...]


def _mlp_pool(target, gath, val, src, W1p, b1r, W2, b2r, W3, b3r):
    n_tgt = target.shape[0]
    n_src, width = src.shape
    d_out = b3r.shape[-1]
    T = _pick_block(n_tgt)
    fixed = lambda *shape: pl.BlockSpec(shape, lambda i: (0,) * len(shape))
    return pl.pallas_call(
        _mlp_body,
        grid=(n_tgt // T,),
        in_specs=[
            pl.BlockSpec((T, 3), lambda i: (i, 0)),
            pl.BlockSpec((_K, T, 128), lambda i: (0, i, 0)),
            pl.BlockSpec((T, _K), lambda i: (i, 0)),
            fixed(n_src, width),
            fixed(3, 128),
            fixed(1, 128),
            fixed(*W2.shape),
            fixed(1, W2.shape[1]),
            fixed(*W3.shape),
            fixed(1, d_out),
        ],
        out_specs=[
            pl.BlockSpec((T, d_out), lambda i: (i, 0)),
            fixed(n_src, width),
        ],
        out_shape=[
            jax.ShapeDtypeStruct((n_tgt, d_out), jnp.float32),
            jax.ShapeDtypeStruct((n_src, width), jnp.float32),
        ],
        scratch_shapes=[pltpu.VMEM((T, 1), jnp.float32)],
    )(target, gath, val, src, W1p, b1r, W2, b2r, W3, b3r)


def _forward(src, target, W1, b1, W2, b2, W3, b3):
    n_tgt = target.shape[0]
    posT = jnp.transpose(src[:, :3])
    srcM = jnp.concatenate([src[:, 3:], src[:, :3]], axis=1)
    W1f = W1[:128, :]
    W1p = W1[128:, :]

    idx, val, F = _select(target, posT, srcM, W1f, W1p)
    idx_kmajor = jnp.transpose(idx).reshape(-1)
    gath_flat = _make_gather(_K * n_tgt, 128)(idx_kmajor, F)
    gath = gath_flat.reshape(_K, n_tgt, 128)

    return _mlp_pool(target, gath, val, src, W1p, b1.reshape(1, -1),
                     W2, b2.reshape(1, -1), W3, b3.reshape(1, -1))


def kernel(src, target, W1, b1, W2, b2, W3, b3):
    _, src_out = _forward(src, target, W1, b1, W2, b2, W3, b3)
    return src_out
